# baseline (speedup 1.0000x reference)
import jax
import jax.numpy as jnp
from jax import lax
from jax.experimental import pallas as pl
from jax.experimental.pallas import tpu as pltpu
from jax.experimental.pallas import tpu_sc as plsc

MIN_VAL = 1e-08
BIG = 1e10
P = 256
L = 19200
B = 8
NSUB = 16
BPQ = L // 4
NV = BPQ // 16
UNROLL = 4
SEGW = 272
NCH = SEGW // 16
SHW = 384
YSHW = 128


def _vsort(v):
    return lax.sort(v, dimension=0)


def _bitonic_merge(vs):
    n = len(vs)
    s = n // 2
    vs = list(vs)
    while s >= 1:
        for i in range(n):
            if (i % (2 * s)) < s:
                a, b = vs[i], vs[i + s]
                vs[i] = jnp.minimum(a, b)
                vs[i + s] = jnp.maximum(a, b)
        s //= 2
    return [_vsort(v) for v in vs]


def _merge_runs(x, y):
    m = len(x)
    y2 = [lax.rev(y[m - 1 - i], (0,)) for i in range(m)]
    lo = [jnp.minimum(x[i], y2[i]) for i in range(m)]
    hi = [jnp.maximum(x[i], y2[i]) for i in range(m)]
    return _bitonic_merge(lo) + _bitonic_merge(hi)


def _regsort256(vs):
    runs = [[_vsort(v)] for v in vs]
    while len(runs) > 1:
        runs = [_merge_runs(runs[i], runs[i + 1]) for i in range(0, len(runs), 2)]
    return runs[0]


def _sc_body(bins_hbm, pts_hbm, out_hbm,
             bins_v, pts_v, ctr_v, segmax_v, segmin_v, red_v,
             cmb_v, pscan_v, sscan_v, ybuf_v, ysh_v, shmax_v, shmin_v, out_v):
    cid = lax.axis_index("c")
    sid = lax.axis_index("s")
    batch = cid * 4 + sid // 4
    quarter = sid % 4

    lanes = lax.iota(jnp.int32, 16)
    negbig = jnp.full((16,), -BIG, jnp.float32)
    posbig = jnp.full((16,), BIG, jnp.float32)

    pltpu.sync_copy(bins_hbm.at[pl.ds(batch * 256, 264)], bins_v)
    pltpu.sync_copy(pts_hbm.at[pl.ds(batch * L + quarter * BPQ, BPQ)], pts_v)

    cvs = []
    for ch in range(16):
        a = plsc.load_gather(bins_v, [lanes + (batch + ch * 16)])
        bb = plsc.load_gather(bins_v, [lanes + (batch + ch * 16 + 1)])
        cvs.append(0.5 * (a + bb))
    svs = _regsort256(cvs)
    for ch in range(16):
        ctr_v[pl.ds(ch * 16, 16)] = svs[ch]

    def _init(ch, _):
        for l in range(16):
            segmax_v[pl.ds(l * SEGW + ch * 16, 16)] = negbig
            segmin_v[pl.ds(l * SEGW + ch * 16, 16)] = posbig
        return 0
    lax.fori_loop(0, NCH, _init, 0)

    c255 = plsc.load_gather(ctr_v, [jnp.full((16,), P - 1, jnp.int32)])

    def _point(i):
        t = pts_v[pl.ds(i * 16, 16)]
        valid = t >= MIN_VAL
        k = jnp.where(c255 <= t, P, 0)
        for step in (128, 64, 32, 16, 8, 4, 2, 1):
            cand = k + step
            idx = jnp.minimum(cand - 1, P - 1)
            cv = plsc.load_gather(ctr_v, [idx])
            ok = (cand <= P) & (cv <= t)
            k = jnp.where(ok, cand, k)
        km1 = jnp.maximum(k - 1, 0)
        kcl = jnp.minimum(k, P - 1)
        c_lo = plsc.load_gather(ctr_v, [km1])
        c_hi = plsc.load_gather(ctr_v, [kcl])
        dd = jnp.where(k >= 1, (t - c_lo) * (t - c_lo), BIG)
        du = jnp.where(k < P, (c_hi - t) * (c_hi - t), BIG)
        dmin = jnp.minimum(dd, du)
        addr = lanes * SEGW + k
        tmx = jnp.where(valid, t, negbig)
        tmn = jnp.where(valid, t, posbig)
        cm = plsc.load_gather(segmax_v, [addr])
        plsc.store_scatter(segmax_v, [addr], jnp.maximum(cm, tmx))
        cn = plsc.load_gather(segmin_v, [addr])
        plsc.store_scatter(segmin_v, [addr], jnp.minimum(cn, tmn))
        return jnp.where(valid, dmin, 0.0), jnp.where(valid, 1.0, 0.0)

    def _pt(i, carry):
        ysum, ycnt = carry
        w0, n0 = _point(i * UNROLL)
        w1, n1 = _point(i * UNROLL + 1)
        w2, n2 = _point(i * UNROLL + 2)
        w3, n3 = _point(i * UNROLL + 3)
        return ysum + ((w0 + w1) + (w2 + w3)), ycnt + ((n0 + n1) + (n2 + n3))

    ysum, ycnt = lax.fori_loop(
        0, NV // UNROLL, _pt,
        (jnp.zeros((16,), jnp.float32), jnp.zeros((16,), jnp.float32)))

    def _red(ch, _):
        accx = negbig
        accn = posbig
        for l in range(16):
            accx = jnp.maximum(accx, segmax_v[pl.ds(l * SEGW + ch * 16, 16)])
            accn = jnp.minimum(accn, segmin_v[pl.ds(l * SEGW + ch * 16, 16)])
        red_v[pl.ds(ch * 16, 16)] = accx
        red_v[pl.ds(SHW + ch * 16, 16)] = accn
        return 0
    lax.fori_loop(0, NCH, _red, 0)

    pltpu.sync_copy(red_v.at[pl.ds(0, SHW)], shmax_v.at[pl.ds(sid * SHW, SHW)])
    pltpu.sync_copy(red_v.at[pl.ds(SHW, SHW)], shmin_v.at[pl.ds(sid * SHW, SHW)])
    ysh_row = jnp.where(lanes < 8,
                        jnp.full((16,), jnp.sum(ysum)),
                        jnp.full((16,), jnp.sum(ycnt)))
    ybuf_v[pl.ds(0, 16)] = ysh_row
    pltpu.sync_copy(ybuf_v, ysh_v.at[pl.ds(sid * YSHW, YSHW)])
    plsc.subcore_barrier()

    @pl.when(quarter == 0)
    def _leader():
        for q in range(4):
            pltpu.sync_copy(shmax_v.at[pl.ds((sid + q) * SHW, SHW)],
                            cmb_v.at[pl.ds(q * SHW, SHW)])
            pltpu.sync_copy(shmin_v.at[pl.ds((sid + q) * SHW, SHW)],
                            cmb_v.at[pl.ds((4 + q) * SHW, SHW)])
            pltpu.sync_copy(ysh_v.at[pl.ds((sid + q) * YSHW, YSHW)],
                            cmb_v.at[pl.ds(8 * SHW + q * YSHW, YSHW)])

        def _pscan(ch, carry):
            v = negbig
            for q in range(4):
                v = jnp.maximum(v, cmb_v[pl.ds(q * SHW + ch * 16, 16)])
            v = jnp.maximum(plsc.cummax(v), jnp.full((16,), carry))
            pscan_v[pl.ds(ch * 16, 16)] = v
            return jnp.max(v)
        lax.fori_loop(0, NCH, _pscan, jnp.float32(-BIG))

        def _sscan(j, carry):
            ch = NCH - 1 - j
            v = posbig
            for q in range(4):
                v = jnp.minimum(v, cmb_v[pl.ds((4 + q) * SHW + ch * 16, 16)])
            rv = lax.rev(v, (0,))
            sfx = lax.rev(-plsc.cummax(-rv), (0,))
            sfx = jnp.minimum(sfx, jnp.full((16,), carry))
            sscan_v[pl.ds(ch * 16, 16)] = sfx
            return jnp.min(sfx)
        lax.fori_loop(0, NCH, _sscan, jnp.float32(BIG))

        def _chx(ch, acc):
            cj = ctr_v[pl.ds(ch * 16, 16)]
            down = pscan_v[pl.ds(ch * 16, 16)]
            up = plsc.load_gather(sscan_v, [lanes + (ch * 16 + 1)])
            d1 = (cj - down) * (cj - down)
            d2 = (up - cj) * (up - cj)
            return acc + jnp.sum(jnp.minimum(jnp.minimum(d1, d2), BIG))
        chx = lax.fori_loop(0, P // 16, _chx, jnp.float32(0.0))

        ys = jnp.float32(0.0)
        yc = jnp.float32(0.0)
        for q in range(4):
            row = cmb_v[pl.ds(8 * SHW + q * YSHW, 16)]
            ys = ys + jnp.sum(jnp.where(lanes == 0, row, 0.0))
            yc = yc + jnp.sum(jnp.where(lanes == 8, row, 0.0))
        ov = jnp.where(lanes == 0, jnp.full((16,), chx), 0.0)
        ov = jnp.where(lanes == 1, jnp.full((16,), ys), ov)
        ov = jnp.where(lanes == 2, jnp.full((16,), yc), ov)
        out_v[...] = ov
        pltpu.sync_copy(out_v, out_hbm.at[pl.ds(batch * 16, 16)])


def _sc_chamfer(bins_flat, t_flat):
    mesh = plsc.VectorSubcoreMesh(core_axis_name="c", subcore_axis_name="s")
    f = pl.kernel(
        _sc_body,
        out_type=jax.ShapeDtypeStruct((B * 16,), jnp.float32),
        mesh=mesh,
        compiler_params=pltpu.CompilerParams(needs_layout_passes=False),
        scratch_types=[
            pltpu.VMEM((264,), jnp.float32),
            pltpu.VMEM((BPQ,), jnp.float32),
            pltpu.VMEM((P,), jnp.float32),
            pltpu.VMEM((16 * SEGW,), jnp.float32),
            pltpu.VMEM((16 * SEGW,), jnp.float32),
            pltpu.VMEM((2 * SHW,), jnp.float32),
            pltpu.VMEM((8 * SHW + 4 * YSHW,), jnp.float32),
            pltpu.VMEM((SEGW,), jnp.float32),
            pltpu.VMEM((SEGW + 16,), jnp.float32),
            pltpu.VMEM((YSHW,), jnp.float32),
            pltpu.VMEM_SHARED((NSUB * YSHW,), jnp.float32),
            pltpu.VMEM_SHARED((NSUB * SHW,), jnp.float32),
            pltpu.VMEM_SHARED((NSUB * SHW,), jnp.float32),
            pltpu.VMEM((16,), jnp.float32),
        ],
    )
    return f(bins_flat, t_flat)


def kernel(bins, target_depth_maps):
    o = _sc_chamfer(bins.reshape(B * (P + 1)),
                    target_depth_maps.reshape(B * L)).reshape(B, 16)
    cham_x = o[:, 0] / jnp.float32(P)
    cham_y = o[:, 1] / jnp.maximum(o[:, 2], 1.0)
    return jnp.sum(cham_x + cham_y) / jnp.float32(B)


# speedup vs baseline: 1.2091x; 1.2091x over previous
import jax
import jax.numpy as jnp
from jax import lax
from jax.experimental import pallas as pl
from jax.experimental.pallas import tpu as pltpu
from jax.experimental.pallas import tpu_sc as plsc

MIN_VAL = 1e-08
BIG = 1e10
P = 256
L = 19200
B = 8
NSUB = 16
BPQ = L // 4
NV = BPQ // 16
UNROLL = 4
SEGW = 272
NCH = SEGW // 16
SHW = 384
YSHW = 128


def _vsort(v):
    return lax.sort(v, dimension=0)


def _bitonic_merge(vs):
    n = len(vs)
    s = n // 2
    vs = list(vs)
    while s >= 1:
        for i in range(n):
            if (i % (2 * s)) < s:
                a, b = vs[i], vs[i + s]
                vs[i] = jnp.minimum(a, b)
                vs[i + s] = jnp.maximum(a, b)
        s //= 2
    return [_vsort(v) for v in vs]


def _merge_runs(x, y):
    m = len(x)
    y2 = [lax.rev(y[m - 1 - i], (0,)) for i in range(m)]
    lo = [jnp.minimum(x[i], y2[i]) for i in range(m)]
    hi = [jnp.maximum(x[i], y2[i]) for i in range(m)]
    return _bitonic_merge(lo) + _bitonic_merge(hi)


def _regsort256(vs):
    runs = [[_vsort(v)] for v in vs]
    while len(runs) > 1:
        runs = [_merge_runs(runs[i], runs[i + 1]) for i in range(0, len(runs), 2)]
    return runs[0]


def _sc_body(bins_hbm, pts_hbm, out_hbm,
             bins_v, pts_v, kidx_v, ctr_v, segmax_v, segmin_v, red_v,
             cmb_v, pscan_v, sscan_v, ybuf_v, ysh_v, shmax_v, shmin_v, out_v):
    cid = lax.axis_index("c")
    sid = lax.axis_index("s")
    batch = cid * 4 + sid // 4
    quarter = sid % 4

    lanes = lax.iota(jnp.int32, 16)
    negbig = jnp.full((16,), -BIG, jnp.float32)
    posbig = jnp.full((16,), BIG, jnp.float32)

    pltpu.sync_copy(bins_hbm.at[pl.ds(batch * 256, 264)], bins_v)
    pltpu.sync_copy(pts_hbm.at[pl.ds(batch * L + quarter * BPQ, BPQ)], pts_v)

    cvs = []
    for ch in range(16):
        a = plsc.load_gather(bins_v, [lanes + (batch + ch * 16)])
        bb = plsc.load_gather(bins_v, [lanes + (batch + ch * 16 + 1)])
        cvs.append(0.5 * (a + bb))
    svs = _regsort256(cvs)
    for ch in range(16):
        ctr_v[pl.ds(ch * 16, 16)] = svs[ch]

    def _init(ch, _):
        for l in range(16):
            segmax_v[pl.ds(l * SEGW + ch * 16, 16)] = negbig
            segmin_v[pl.ds(l * SEGW + ch * 16, 16)] = posbig
        return 0
    lax.fori_loop(0, NCH, _init, 0)

    c255 = plsc.load_gather(ctr_v, [jnp.full((16,), P - 1, jnp.int32)])

    @plsc.parallel_loop(0, NV, unroll=UNROLL,
                        carry=(jnp.zeros((16,), jnp.float32),
                               jnp.zeros((16,), jnp.float32)))
    def _pass1(i, carry):
        ysum, ycnt = carry
        t = pts_v[pl.ds(i * 16, 16)]
        valid = t >= MIN_VAL
        k = jnp.where(c255 <= t, P, 0)
        c_lo = c255
        c_hi = c255
        for step in (128, 64, 32, 16, 8, 4, 2, 1):
            cand = k + step
            idx = jnp.minimum(cand - 1, P - 1)
            cv = plsc.load_gather(ctr_v, [idx])
            ok = (cand <= P) & (cv <= t)
            k = jnp.where(ok, cand, k)
            c_lo = jnp.where(ok, cv, c_lo)
            c_hi = jnp.where(ok, c_hi, cv)
        kidx_v[pl.ds(i * 16, 16)] = k
        dd = jnp.where(k >= 1, (t - c_lo) * (t - c_lo), BIG)
        du = jnp.where(k < P, (c_hi - t) * (c_hi - t), BIG)
        dmin = jnp.minimum(dd, du)
        return (ysum + jnp.where(valid, dmin, 0.0),
                ycnt + jnp.where(valid, 1.0, 0.0))

    ysum, ycnt = _pass1

    def _pass2(i, _):
        t = pts_v[pl.ds(i * 16, 16)]
        k = kidx_v[pl.ds(i * 16, 16)]
        valid = t >= MIN_VAL
        addr = lanes * SEGW + k
        tmx = jnp.where(valid, t, negbig)
        tmn = jnp.where(valid, t, posbig)
        cm = plsc.load_gather(segmax_v, [addr])
        plsc.store_scatter(segmax_v, [addr], jnp.maximum(cm, tmx))
        cn = plsc.load_gather(segmin_v, [addr])
        plsc.store_scatter(segmin_v, [addr], jnp.minimum(cn, tmn))
        return 0
    lax.fori_loop(0, NV, _pass2, 0)

    def _red(ch, _):
        accx = negbig
        accn = posbig
        for l in range(16):
            accx = jnp.maximum(accx, segmax_v[pl.ds(l * SEGW + ch * 16, 16)])
            accn = jnp.minimum(accn, segmin_v[pl.ds(l * SEGW + ch * 16, 16)])
        red_v[pl.ds(ch * 16, 16)] = accx
        red_v[pl.ds(SHW + ch * 16, 16)] = accn
        return 0
    lax.fori_loop(0, NCH, _red, 0)

    pltpu.sync_copy(red_v.at[pl.ds(0, SHW)], shmax_v.at[pl.ds(sid * SHW, SHW)])
    pltpu.sync_copy(red_v.at[pl.ds(SHW, SHW)], shmin_v.at[pl.ds(sid * SHW, SHW)])
    ysh_row = jnp.where(lanes < 8,
                        jnp.full((16,), jnp.sum(ysum)),
                        jnp.full((16,), jnp.sum(ycnt)))
    ybuf_v[pl.ds(0, 16)] = ysh_row
    pltpu.sync_copy(ybuf_v, ysh_v.at[pl.ds(sid * YSHW, YSHW)])
    plsc.subcore_barrier()

    @pl.when(quarter == 0)
    def _leader():
        for q in range(4):
            pltpu.sync_copy(shmax_v.at[pl.ds((sid + q) * SHW, SHW)],
                            cmb_v.at[pl.ds(q * SHW, SHW)])
            pltpu.sync_copy(shmin_v.at[pl.ds((sid + q) * SHW, SHW)],
                            cmb_v.at[pl.ds((4 + q) * SHW, SHW)])
            pltpu.sync_copy(ysh_v.at[pl.ds((sid + q) * YSHW, YSHW)],
                            cmb_v.at[pl.ds(8 * SHW + q * YSHW, YSHW)])

        def _pscan(ch, carry):
            v = negbig
            for q in range(4):
                v = jnp.maximum(v, cmb_v[pl.ds(q * SHW + ch * 16, 16)])
            v = jnp.maximum(plsc.cummax(v), jnp.full((16,), carry))
            pscan_v[pl.ds(ch * 16, 16)] = v
            return jnp.max(v)
        lax.fori_loop(0, NCH, _pscan, jnp.float32(-BIG))

        def _sscan(j, carry):
            ch = NCH - 1 - j
            v = posbig
            for q in range(4):
                v = jnp.minimum(v, cmb_v[pl.ds((4 + q) * SHW + ch * 16, 16)])
            rv = lax.rev(v, (0,))
            sfx = lax.rev(-plsc.cummax(-rv), (0,))
            sfx = jnp.minimum(sfx, jnp.full((16,), carry))
            sscan_v[pl.ds(ch * 16, 16)] = sfx
            return jnp.min(sfx)
        lax.fori_loop(0, NCH, _sscan, jnp.float32(BIG))

        def _chx(ch, acc):
            cj = ctr_v[pl.ds(ch * 16, 16)]
            down = pscan_v[pl.ds(ch * 16, 16)]
            up = plsc.load_gather(sscan_v, [lanes + (ch * 16 + 1)])
            d1 = (cj - down) * (cj - down)
            d2 = (up - cj) * (up - cj)
            return acc + jnp.sum(jnp.minimum(jnp.minimum(d1, d2), BIG))
        chx = lax.fori_loop(0, P // 16, _chx, jnp.float32(0.0))

        ys = jnp.float32(0.0)
        yc = jnp.float32(0.0)
        for q in range(4):
            row = cmb_v[pl.ds(8 * SHW + q * YSHW, 16)]
            ys = ys + jnp.sum(jnp.where(lanes == 0, row, 0.0))
            yc = yc + jnp.sum(jnp.where(lanes == 8, row, 0.0))
        ov = jnp.where(lanes == 0, jnp.full((16,), chx), 0.0)
        ov = jnp.where(lanes == 1, jnp.full((16,), ys), ov)
        ov = jnp.where(lanes == 2, jnp.full((16,), yc), ov)
        out_v[...] = ov
        pltpu.sync_copy(out_v, out_hbm.at[pl.ds(batch * 16, 16)])


def _sc_chamfer(bins_flat, t_flat):
    mesh = plsc.VectorSubcoreMesh(core_axis_name="c", subcore_axis_name="s")
    f = pl.kernel(
        _sc_body,
        out_type=jax.ShapeDtypeStruct((B * 16,), jnp.float32),
        mesh=mesh,
        compiler_params=pltpu.CompilerParams(needs_layout_passes=False),
        scratch_types=[
            pltpu.VMEM((264,), jnp.float32),
            pltpu.VMEM((BPQ,), jnp.float32),
            pltpu.VMEM((BPQ,), jnp.int32),
            pltpu.VMEM((P,), jnp.float32),
            pltpu.VMEM((16 * SEGW,), jnp.float32),
            pltpu.VMEM((16 * SEGW,), jnp.float32),
            pltpu.VMEM((2 * SHW,), jnp.float32),
            pltpu.VMEM((8 * SHW + 4 * YSHW,), jnp.float32),
            pltpu.VMEM((SEGW,), jnp.float32),
            pltpu.VMEM((SEGW + 16,), jnp.float32),
            pltpu.VMEM((YSHW,), jnp.float32),
            pltpu.VMEM_SHARED((NSUB * YSHW,), jnp.float32),
            pltpu.VMEM_SHARED((NSUB * SHW,), jnp.float32),
            pltpu.VMEM_SHARED((NSUB * SHW,), jnp.float32),
            pltpu.VMEM((16,), jnp.float32),
        ],
    )
    return f(bins_flat, t_flat)


def kernel(bins, target_depth_maps):
    o = _sc_chamfer(bins.reshape(B * (P + 1)),
                    target_depth_maps.reshape(B * L)).reshape(B, 16)
    cham_x = o[:, 0] / jnp.float32(P)
    cham_y = o[:, 1] / jnp.maximum(o[:, 2], 1.0)
    return jnp.sum(cham_x + cham_y) / jnp.float32(B)


# speedup vs baseline: 1.2796x; 1.0583x over previous
import jax
import jax.numpy as jnp
from jax import lax
from jax.experimental import pallas as pl
from jax.experimental.pallas import tpu as pltpu
from jax.experimental.pallas import tpu_sc as plsc

MIN_VAL = 1e-08
BIG = 1e10
P = 256
L = 19200
B = 8
NSUB = 16
BPQ = L // 4
NV = BPQ // 16
UNROLL = 8
SEGW = 272
NCH = SEGW // 16
SHW = 384
YSHW = 128


def _vsort(v):
    return lax.sort(v, dimension=0)


def _bitonic_merge(vs):
    n = len(vs)
    s = n // 2
    vs = list(vs)
    while s >= 1:
        for i in range(n):
            if (i % (2 * s)) < s:
                a, b = vs[i], vs[i + s]
                vs[i] = jnp.minimum(a, b)
                vs[i + s] = jnp.maximum(a, b)
        s //= 2
    return [_vsort(v) for v in vs]


def _merge_runs(x, y):
    m = len(x)
    y2 = [lax.rev(y[m - 1 - i], (0,)) for i in range(m)]
    lo = [jnp.minimum(x[i], y2[i]) for i in range(m)]
    hi = [jnp.maximum(x[i], y2[i]) for i in range(m)]
    return _bitonic_merge(lo) + _bitonic_merge(hi)


def _regsort256(vs):
    runs = [[_vsort(v)] for v in vs]
    while len(runs) > 1:
        runs = [_merge_runs(runs[i], runs[i + 1]) for i in range(0, len(runs), 2)]
    return runs[0]


def _sc_body(bins_hbm, pts_hbm, out_hbm,
             bins_v, pts_v, kidx_v, ctr_v, segmax_v, segmin_v, red_v,
             cmb_v, pscan_v, sscan_v, ybuf_v, ysh_v, shmax_v, shmin_v, out_v):
    cid = lax.axis_index("c")
    sid = lax.axis_index("s")
    batch = cid * 4 + sid // 4
    quarter = sid % 4

    lanes = lax.iota(jnp.int32, 16)
    negbig = jnp.full((16,), -BIG, jnp.float32)
    posbig = jnp.full((16,), BIG, jnp.float32)

    pltpu.sync_copy(bins_hbm.at[pl.ds(batch * 256, 264)], bins_v)
    pltpu.sync_copy(pts_hbm.at[pl.ds(batch * L + quarter * BPQ, BPQ)], pts_v)

    cvs = []
    for ch in range(16):
        a = plsc.load_gather(bins_v, [lanes + (batch + ch * 16)])
        bb = plsc.load_gather(bins_v, [lanes + (batch + ch * 16 + 1)])
        cvs.append(0.5 * (a + bb))
    svs = _regsort256(cvs)
    for ch in range(16):
        ctr_v[pl.ds(ch * 16, 16)] = svs[ch]

    def _init(ch, _):
        for l in range(16):
            segmax_v[pl.ds(l * SEGW + ch * 16, 16)] = negbig
            segmin_v[pl.ds(l * SEGW + ch * 16, 16)] = posbig
        return 0
    lax.fori_loop(0, NCH, _init, 0)

    c255 = plsc.load_gather(ctr_v, [jnp.full((16,), P - 1, jnp.int32)])

    @plsc.parallel_loop(0, NV, unroll=UNROLL,
                        carry=(jnp.zeros((16,), jnp.float32),
                               jnp.zeros((16,), jnp.float32)))
    def _pass1(i, carry):
        ysum, ycnt = carry
        t = pts_v[pl.ds(i * 16, 16)]
        valid = t >= MIN_VAL
        k = jnp.where(c255 <= t, P, 0)
        c_lo = c255
        c_hi = c255
        for step in (128, 64, 32, 16, 8, 4, 2, 1):
            cand = k + step
            idx = jnp.minimum(cand - 1, P - 1)
            cv = plsc.load_gather(ctr_v, [idx])
            ok = (cand <= P) & (cv <= t)
            k = jnp.where(ok, cand, k)
            c_lo = jnp.where(ok, cv, c_lo)
            c_hi = jnp.where(ok, c_hi, cv)
        kidx_v[pl.ds(i * 16, 16)] = k
        dd = jnp.where(k >= 1, (t - c_lo) * (t - c_lo), BIG)
        du = jnp.where(k < P, (c_hi - t) * (c_hi - t), BIG)
        dmin = jnp.minimum(dd, du)
        return (ysum + jnp.where(valid, dmin, 0.0),
                ycnt + jnp.where(valid, 1.0, 0.0))

    ysum, ycnt = _pass1

    def _pass2(i, _):
        t = pts_v[pl.ds(i * 16, 16)]
        k = kidx_v[pl.ds(i * 16, 16)]
        valid = t >= MIN_VAL
        addr = lanes * SEGW + k
        tmx = jnp.where(valid, t, negbig)
        tmn = jnp.where(valid, t, posbig)
        cm = plsc.load_gather(segmax_v, [addr])
        plsc.store_scatter(segmax_v, [addr], jnp.maximum(cm, tmx))
        cn = plsc.load_gather(segmin_v, [addr])
        plsc.store_scatter(segmin_v, [addr], jnp.minimum(cn, tmn))
        return 0
    lax.fori_loop(0, NV, _pass2, 0)

    def _red(ch, _):
        accx = negbig
        accn = posbig
        for l in range(16):
            accx = jnp.maximum(accx, segmax_v[pl.ds(l * SEGW + ch * 16, 16)])
            accn = jnp.minimum(accn, segmin_v[pl.ds(l * SEGW + ch * 16, 16)])
        red_v[pl.ds(ch * 16, 16)] = accx
        red_v[pl.ds(SHW + ch * 16, 16)] = accn
        return 0
    lax.fori_loop(0, NCH, _red, 0)

    pltpu.sync_copy(red_v.at[pl.ds(0, SHW)], shmax_v.at[pl.ds(sid * SHW, SHW)])
    pltpu.sync_copy(red_v.at[pl.ds(SHW, SHW)], shmin_v.at[pl.ds(sid * SHW, SHW)])
    ysh_row = jnp.where(lanes < 8,
                        jnp.full((16,), jnp.sum(ysum)),
                        jnp.full((16,), jnp.sum(ycnt)))
    ybuf_v[pl.ds(0, 16)] = ysh_row
    pltpu.sync_copy(ybuf_v, ysh_v.at[pl.ds(sid * YSHW, YSHW)])
    plsc.subcore_barrier()

    @pl.when(quarter == 0)
    def _leader():
        for q in range(4):
            pltpu.sync_copy(shmax_v.at[pl.ds((sid + q) * SHW, SHW)],
                            cmb_v.at[pl.ds(q * SHW, SHW)])
            pltpu.sync_copy(shmin_v.at[pl.ds((sid + q) * SHW, SHW)],
                            cmb_v.at[pl.ds((4 + q) * SHW, SHW)])
            pltpu.sync_copy(ysh_v.at[pl.ds((sid + q) * YSHW, YSHW)],
                            cmb_v.at[pl.ds(8 * SHW + q * YSHW, YSHW)])

        def _pscan(ch, carry):
            v = negbig
            for q in range(4):
                v = jnp.maximum(v, cmb_v[pl.ds(q * SHW + ch * 16, 16)])
            v = jnp.maximum(plsc.cummax(v), jnp.full((16,), carry))
            pscan_v[pl.ds(ch * 16, 16)] = v
            return jnp.max(v)
        lax.fori_loop(0, NCH, _pscan, jnp.float32(-BIG))

        def _sscan(j, carry):
            ch = NCH - 1 - j
            v = posbig
            for q in range(4):
                v = jnp.minimum(v, cmb_v[pl.ds((4 + q) * SHW + ch * 16, 16)])
            rv = lax.rev(v, (0,))
            sfx = lax.rev(-plsc.cummax(-rv), (0,))
            sfx = jnp.minimum(sfx, jnp.full((16,), carry))
            sscan_v[pl.ds(ch * 16, 16)] = sfx
            return jnp.min(sfx)
        lax.fori_loop(0, NCH, _sscan, jnp.float32(BIG))

        def _chx(ch, acc):
            cj = ctr_v[pl.ds(ch * 16, 16)]
            down = pscan_v[pl.ds(ch * 16, 16)]
            up = plsc.load_gather(sscan_v, [lanes + (ch * 16 + 1)])
            d1 = (cj - down) * (cj - down)
            d2 = (up - cj) * (up - cj)
            return acc + jnp.sum(jnp.minimum(jnp.minimum(d1, d2), BIG))
        chx = lax.fori_loop(0, P // 16, _chx, jnp.float32(0.0))

        ys = jnp.float32(0.0)
        yc = jnp.float32(0.0)
        for q in range(4):
            row = cmb_v[pl.ds(8 * SHW + q * YSHW, 16)]
            ys = ys + jnp.sum(jnp.where(lanes == 0, row, 0.0))
            yc = yc + jnp.sum(jnp.where(lanes == 8, row, 0.0))
        ov = jnp.where(lanes == 0, jnp.full((16,), chx), 0.0)
        ov = jnp.where(lanes == 1, jnp.full((16,), ys), ov)
        ov = jnp.where(lanes == 2, jnp.full((16,), yc), ov)
        out_v[...] = ov
        pltpu.sync_copy(out_v, out_hbm.at[pl.ds(batch * 16, 16)])


def _sc_chamfer(bins_flat, t_flat):
    mesh = plsc.VectorSubcoreMesh(core_axis_name="c", subcore_axis_name="s")
    f = pl.kernel(
        _sc_body,
        out_type=jax.ShapeDtypeStruct((B * 16,), jnp.float32),
        mesh=mesh,
        compiler_params=pltpu.CompilerParams(needs_layout_passes=False),
        scratch_types=[
            pltpu.VMEM((264,), jnp.float32),
            pltpu.VMEM((BPQ,), jnp.float32),
            pltpu.VMEM((BPQ,), jnp.int32),
            pltpu.VMEM((P,), jnp.float32),
            pltpu.VMEM((16 * SEGW,), jnp.float32),
            pltpu.VMEM((16 * SEGW,), jnp.float32),
            pltpu.VMEM((2 * SHW,), jnp.float32),
            pltpu.VMEM((8 * SHW + 4 * YSHW,), jnp.float32),
            pltpu.VMEM((SEGW,), jnp.float32),
            pltpu.VMEM((SEGW + 16,), jnp.float32),
            pltpu.VMEM((YSHW,), jnp.float32),
            pltpu.VMEM_SHARED((NSUB * YSHW,), jnp.float32),
            pltpu.VMEM_SHARED((NSUB * SHW,), jnp.float32),
            pltpu.VMEM_SHARED((NSUB * SHW,), jnp.float32),
            pltpu.VMEM((16,), jnp.float32),
        ],
    )
    return f(bins_flat, t_flat)


def kernel(bins, target_depth_maps):
    o = _sc_chamfer(bins.reshape(B * (P + 1)),
                    target_depth_maps.reshape(B * L)).reshape(B, 16)
    cham_x = o[:, 0] / jnp.float32(P)
    cham_y = o[:, 1] / jnp.maximum(o[:, 2], 1.0)
    return jnp.sum(cham_x + cham_y) / jnp.float32(B)
